# baseline (speedup 1.0000x reference)
import jax
import jax.numpy as jnp
from jax.experimental import pallas as pl

N = 10000
H = 128


def _ident_kernel(x_ref, o_ref):
    o_ref[...] = x_ref[...]


def _conv(x, src, dst, Wq, bq, Wk, bk, Wv, bv, Ws, bs):
    q = x @ Wq + bq
    k = x @ Wk + bk
    v = x @ Wv + bv
    logits = jnp.sum(q[dst] * k[src], axis=-1) / jnp.sqrt(jnp.float32(H))
    m = jax.ops.segment_max(logits, dst, num_segments=N)
    m = jnp.where(jnp.isfinite(m), m, 0.0)
    e = jnp.exp(logits - m[dst])
    s = jax.ops.segment_sum(e, dst, num_segments=N)
    alpha = e / (s[dst] + 1e-16)
    out = jax.ops.segment_sum(alpha[:, None] * v[src], dst, num_segments=N)
    return out + (x @ Ws + bs)


def kernel(x, edge_index, Wq1, bq1, Wk1, bk1, Wv1, bv1, Ws1, bs1,
           Wq2, bq2, Wk2, bk2, Wv2, bv2, Ws2, bs2):
    src = edge_index[0]
    dst = edge_index[1]
    h = _conv(x, src, dst, Wq1, bq1, Wk1, bk1, Wv1, bv1, Ws1, bs1)
    h = jax.nn.relu(h)
    h = _conv(h, src, dst, Wq2, bq2, Wk2, bk2, Wv2, bv2, Ws2, bs2)
    h = pl.pallas_call(
        _ident_kernel,
        out_shape=jax.ShapeDtypeStruct(h.shape, h.dtype),
    )(h)
    return h


# speedup vs baseline: 3.6727x; 3.6727x over previous
import functools

import jax
import jax.numpy as jnp
import numpy as np
from jax import lax
from jax.experimental import pallas as pl
from jax.experimental.pallas import tpu as pltpu
from jax.experimental.pallas import tpu_sc as plsc

N = 10000
E = 320000
D = 128
H = 128

NC = 2
NS = 16
NW = NC * NS
NP = 10112
EWP = 10240
EP = EWP * NW
CH = 128
NCH = EWP // CH
ROWS_W = NP // NS
SEG16 = NP // 16
INV_SQRT_H = float(1.0 / np.sqrt(H))

_mesh = plsc.VectorSubcoreMesh(core_axis_name="c", subcore_axis_name="s")


def _worker_id():
    return lax.axis_index("c") * NS + lax.axis_index("s")


def _global_max(maxes_vm):
    rmax = maxes_vm[pl.ds(0, 16)]
    for i in range(1, NW):
        rmax = jnp.maximum(rmax, maxes_vm[pl.ds(i * 16, 16)])
    return jnp.max(rmax)


@functools.partial(
    pl.kernel,
    out_type=[
        jax.ShapeDtypeStruct((EP,), jnp.float32),
        jax.ShapeDtypeStruct((NW, 16), jnp.float32),
    ],
    mesh=_mesh,
    compiler_params=pltpu.CompilerParams(needs_layout_passes=False),
    scratch_types=[
        pltpu.VMEM((CH,), jnp.int32),
        pltpu.VMEM((CH,), jnp.int32),
        pltpu.VMEM((CH, D), jnp.float32),
        pltpu.VMEM((CH, D), jnp.float32),
        pltpu.VMEM((CH,), jnp.float32),
        pltpu.VMEM((16,), jnp.float32),
        pltpu.SemaphoreType.DMA,
        pltpu.SemaphoreType.DMA,
    ],
)
def _sc_logits(q_hbm, k_hbm, src_hbm, dst_hbm, logits_hbm, maxes_hbm,
               dsti, srci, qrows, krows, lchunk, mxbuf, sem1, sem2):
    wid = _worker_id()
    base = wid * EWP
    lanes = lax.iota(jnp.int32, 16)
    inv = jnp.float32(INV_SQRT_H)

    def chunk_body(t, rmax):
        off = base + t * CH
        pltpu.sync_copy(dst_hbm.at[pl.ds(off, CH)], dsti)
        pltpu.sync_copy(src_hbm.at[pl.ds(off, CH)], srci)
        cp1 = pltpu.async_copy(q_hbm.at[dsti], qrows, sem1)
        cp2 = pltpu.async_copy(k_hbm.at[srci], krows, sem2)
        cp1.wait()
        cp2.wait()

        def group_body(g, rmax):
            lg = jnp.zeros((16,), jnp.float32)
            for j in range(16):
                r = g * 16 + j
                acc = qrows[r, pl.ds(0, 16)] * krows[r, pl.ds(0, 16)]
                for b in range(1, D // 16):
                    sl = pl.ds(b * 16, 16)
                    acc = acc + qrows[r, sl] * krows[r, sl]
                dv = jnp.sum(acc)
                lg = jnp.where(lanes == j, dv, lg)
            lg = lg * inv
            lchunk[pl.ds(g * 16, 16)] = lg
            return jnp.maximum(rmax, lg)

        rmax = lax.fori_loop(0, CH // 16, group_body, rmax)
        pltpu.sync_copy(lchunk, logits_hbm.at[pl.ds(off, CH)])
        return rmax

    rmax = lax.fori_loop(0, NCH, chunk_body,
                         jnp.full((16,), -1e30, jnp.float32))
    mxbuf[...] = rmax
    pltpu.sync_copy(mxbuf, maxes_hbm.at[wid])


@functools.partial(
    pl.kernel,
    out_type=jax.ShapeDtypeStruct((NC, NP), jnp.float32),
    mesh=_mesh,
    compiler_params=pltpu.CompilerParams(needs_layout_passes=False),
    scratch_types=[
        pltpu.VMEM_SHARED((NP,), jnp.float32),
        pltpu.VMEM((NP,), jnp.float32),
        pltpu.VMEM((NW * 16,), jnp.float32),
        pltpu.VMEM((CH,), jnp.int32),
        pltpu.VMEM((CH,), jnp.float32),
        pltpu.VMEM((CH,), jnp.float32),
    ],
)
def _sc_sums(logits_hbm, dst_hbm, maxes_hbm, sums_hbm,
             s_acc, zvm, maxes_vm, dsti, lchunk, echunk):
    cid = lax.axis_index("c")
    sid = lax.axis_index("s")
    wid = cid * NS + sid
    base = wid * EWP

    pltpu.sync_copy(maxes_hbm, maxes_vm)
    gmax = _global_max(maxes_vm)

    @pl.when(sid == 0)
    def _zero():
        z16 = jnp.zeros((16,), jnp.float32)

        def zb(i, carry):
            zvm[pl.ds(i * 16, 16)] = z16
            return carry

        lax.fori_loop(0, SEG16, zb, 0)
        pltpu.sync_copy(zvm, s_acc)

    plsc.subcore_barrier()

    def chunk_body(t, carry):
        off = base + t * CH
        pltpu.sync_copy(dst_hbm.at[pl.ds(off, CH)], dsti)
        pltpu.sync_copy(logits_hbm.at[pl.ds(off, CH)], lchunk)
        for g in range(CH // 16):
            lv = lchunk[pl.ds(g * 16, 16)]
            echunk[pl.ds(g * 16, 16)] = jnp.exp(lv - gmax)
        pltpu.sync_copy(echunk, s_acc.at[dsti], add=True)
        return carry

    lax.fori_loop(0, NCH, chunk_body, 0)
    plsc.subcore_barrier()

    @pl.when(sid == 0)
    def _dump():
        pltpu.sync_copy(s_acc, zvm)
        pltpu.sync_copy(zvm, sums_hbm.at[cid])


@functools.partial(
    pl.kernel,
    out_type=jax.ShapeDtypeStruct((NC, NP, D), jnp.float32),
    mesh=_mesh,
    compiler_params=pltpu.CompilerParams(needs_layout_passes=False),
    scratch_types=[
        pltpu.VMEM_SHARED((NP, D), jnp.float32),
        pltpu.VMEM((NP,), jnp.float32),
        pltpu.VMEM((NW * 16,), jnp.float32),
        pltpu.VMEM((CH,), jnp.int32),
        pltpu.VMEM((CH,), jnp.int32),
        pltpu.VMEM((CH,), jnp.float32),
        pltpu.VMEM((CH,), jnp.float32),
        pltpu.VMEM((CH, D), jnp.float32),
        pltpu.SemaphoreType.DMA,
    ],
)
def _sc_aggregate(v_hbm, src_hbm, dst_hbm, logits_hbm, maxes_hbm, sums_hbm,
                  parts_hbm,
                  out_acc, s_full, maxes_vm, dsti, srci,
                  lchunk, alpha, vrows, sem):
    cid = lax.axis_index("c")
    sid = lax.axis_index("s")
    wid = cid * NS + sid
    base = wid * EWP

    pltpu.sync_copy(maxes_hbm, maxes_vm)
    gmax = _global_max(maxes_vm)

    pltpu.sync_copy(sums_hbm.at[0], s_full)
    eps = jnp.float32(1e-16)

    def comb(i, carry):
        pltpu.sync_copy(sums_hbm.at[1, pl.ds(i * CH, CH)], alpha)
        for g in range(CH // 16):
            sl = pl.ds(i * CH + g * 16, 16)
            s_full[sl] = s_full[sl] + alpha[pl.ds(g * 16, 16)] + eps
        return carry

    lax.fori_loop(0, NP // CH, comb, 0)

    @pl.when(sid == 0)
    def _zero():
        z16 = jnp.zeros((16,), jnp.float32)

        def zb(i, carry):
            for b in range(D // 16):
                vrows[i, pl.ds(b * 16, 16)] = z16
            return carry

        lax.fori_loop(0, 128, zb, 0)

        def zcopy(i, carry):
            pltpu.sync_copy(vrows, out_acc.at[pl.ds(i * 128, 128)])
            return carry

        lax.fori_loop(0, NP // 128, zcopy, 0)

    plsc.subcore_barrier()

    def chunk_body(t, carry):
        off = base + t * CH
        pltpu.sync_copy(dst_hbm.at[pl.ds(off, CH)], dsti)
        pltpu.sync_copy(src_hbm.at[pl.ds(off, CH)], srci)
        pltpu.sync_copy(logits_hbm.at[pl.ds(off, CH)], lchunk)
        cp = pltpu.async_copy(v_hbm.at[srci], vrows, sem)
        for g in range(CH // 16):
            sl = pl.ds(g * 16, 16)
            ev = jnp.exp(lchunk[sl] - gmax)
            sv = plsc.load_gather(s_full, [dsti[sl]])
            alpha[sl] = ev / sv
        cp.wait()

        def group_scale(g, carry):
            a16 = alpha[pl.ds(g * 16, 16)]
            for j in range(16):
                a = a16[j]
                for b in range(D // 16):
                    sl = pl.ds(b * 16, 16)
                    vrows[g * 16 + j, sl] = vrows[g * 16 + j, sl] * a
            return carry

        lax.fori_loop(0, CH // 16, group_scale, 0)
        pltpu.sync_copy(vrows, out_acc.at[dsti], add=True)
        return carry

    lax.fori_loop(0, NCH, chunk_body, 0)
    plsc.subcore_barrier()

    pltpu.sync_copy(out_acc.at[pl.ds(sid * ROWS_W, ROWS_W)],
                    parts_hbm.at[cid, pl.ds(sid * ROWS_W, ROWS_W)])


_BLK = 2528


def _proj1_body(x_ref, w_ref, b_ref, q_ref, k_ref, v_ref, s_ref):
    r = jnp.dot(x_ref[...], w_ref[...],
                preferred_element_type=jnp.float32) + b_ref[...]
    q_ref[...] = r[:, 0 * D:1 * D]
    k_ref[...] = r[:, 1 * D:2 * D]
    v_ref[...] = r[:, 2 * D:3 * D]
    s_ref[...] = r[:, 3 * D:4 * D]


def _proj2_body(p0_ref, p1_ref, sk_ref, w_ref, b_ref,
                q_ref, k_ref, v_ref, s_ref):
    h = jnp.maximum(p0_ref[...] + p1_ref[...] + sk_ref[...], 0.0)
    r = jnp.dot(h, w_ref[...], preferred_element_type=jnp.float32) + b_ref[...]
    q_ref[...] = r[:, 0 * D:1 * D]
    k_ref[...] = r[:, 1 * D:2 * D]
    v_ref[...] = r[:, 2 * D:3 * D]
    s_ref[...] = r[:, 3 * D:4 * D]


def _combine_body(p0_ref, p1_ref, sk_ref, o_ref):
    o_ref[...] = p0_ref[...] + p1_ref[...] + sk_ref[...]


def _proj1(x, wcat, bcat):
    return pl.pallas_call(
        _proj1_body,
        grid=(NP // _BLK,),
        in_specs=[
            pl.BlockSpec((_BLK, D), lambda i: (i, 0)),
            pl.BlockSpec((D, 4 * D), lambda i: (0, 0)),
            pl.BlockSpec((1, 4 * D), lambda i: (0, 0)),
        ],
        out_specs=[pl.BlockSpec((_BLK, D), lambda i: (i, 0))] * 4,
        out_shape=[jax.ShapeDtypeStruct((NP, D), jnp.float32)] * 4,
    )(x, wcat, bcat)


def _proj2(p0, p1, sk, wcat, bcat):
    return pl.pallas_call(
        _proj2_body,
        grid=(NP // _BLK,),
        in_specs=[
            pl.BlockSpec((_BLK, D), lambda i: (i, 0)),
            pl.BlockSpec((_BLK, D), lambda i: (i, 0)),
            pl.BlockSpec((_BLK, D), lambda i: (i, 0)),
            pl.BlockSpec((D, 4 * D), lambda i: (0, 0)),
            pl.BlockSpec((1, 4 * D), lambda i: (0, 0)),
        ],
        out_specs=[pl.BlockSpec((_BLK, D), lambda i: (i, 0))] * 4,
        out_shape=[jax.ShapeDtypeStruct((NP, D), jnp.float32)] * 4,
    )(p0, p1, sk, wcat, bcat)


def _combine(p0, p1, sk):
    return pl.pallas_call(
        _combine_body,
        grid=(5,),
        in_specs=[pl.BlockSpec((N // 5, D), lambda i: (i, 0))] * 3,
        out_specs=pl.BlockSpec((N // 5, D), lambda i: (i, 0)),
        out_shape=jax.ShapeDtypeStruct((N, D), jnp.float32),
    )(p0, p1, sk)


def _edge_softmax_aggregate(q, k, v, srcp, dstp):
    logits, maxes = _sc_logits(q, k, srcp, dstp)
    sums = _sc_sums(logits, dstp, maxes.reshape(-1))
    parts = _sc_aggregate(v, srcp, dstp, logits, maxes.reshape(-1), sums)
    return parts[0], parts[1]


def kernel(x, edge_index, Wq1, bq1, Wk1, bk1, Wv1, bv1, Ws1, bs1,
           Wq2, bq2, Wk2, bk2, Wv2, bv2, Ws2, bs2):
    src = edge_index[0]
    dst = edge_index[1]
    srcp = jnp.concatenate([src, jnp.zeros((EP - E,), jnp.int32)])
    dstp = jnp.concatenate([dst, jnp.full((EP - E,), N, jnp.int32)])
    xp = jnp.pad(x, ((0, NP - N), (0, 0)))

    w1 = jnp.concatenate([Wq1, Wk1, Wv1, Ws1], axis=1)
    b1 = jnp.concatenate([bq1, bk1, bv1, bs1]).reshape(1, -1)
    w2 = jnp.concatenate([Wq2, Wk2, Wv2, Ws2], axis=1)
    b2 = jnp.concatenate([bq2, bk2, bv2, bs2]).reshape(1, -1)

    q1, k1, v1, sk1 = _proj1(xp, w1, b1)
    p0, p1 = _edge_softmax_aggregate(q1, k1, v1, srcp, dstp)
    q2, k2, v2, sk2 = _proj2(p0, p1, sk1, w2, b2)
    p0, p1 = _edge_softmax_aggregate(q2, k2, v2, srcp, dstp)
    return _combine(p0[:N], p1[:N], sk2[:N])


# speedup vs baseline: 6.2179x; 1.6930x over previous
import functools

import jax
import jax.numpy as jnp
import numpy as np
from jax import lax
from jax.experimental import pallas as pl
from jax.experimental.pallas import tpu as pltpu
from jax.experimental.pallas import tpu_sc as plsc

N = 10000
E = 320000
D = 128
H = 128

NC = 2
NS = 16
NW = NC * NS
NP = 10112
EWP = 10240
EP = EWP * NW
CH = 128
NCH = EWP // CH
ROWS_W = NP // NS
INV_SQRT_H = float(1.0 / np.sqrt(H))

_mesh = plsc.VectorSubcoreMesh(core_axis_name="c", subcore_axis_name="s")
_params = pltpu.CompilerParams(needs_layout_passes=False)


def _worker_id():
    return lax.axis_index("c") * NS + lax.axis_index("s")


def _global_max(maxes_vm):
    rmax = maxes_vm[pl.ds(0, 16)]
    for i in range(1, NW):
        rmax = jnp.maximum(rmax, maxes_vm[pl.ds(i * 16, 16)])
    return jnp.max(rmax)


@functools.partial(
    pl.kernel,
    out_type=[
        jax.ShapeDtypeStruct((EP,), jnp.float32),
        jax.ShapeDtypeStruct((NW, 16), jnp.float32),
    ],
    mesh=_mesh,
    compiler_params=_params,
    scratch_types=[
        pltpu.VMEM((EWP,), jnp.int32),
        pltpu.VMEM((EWP,), jnp.int32),
        pltpu.VMEM((EWP,), jnp.float32),
        pltpu.VMEM((CH, D), jnp.float32),
        pltpu.VMEM((CH, D), jnp.float32),
        pltpu.VMEM((CH, D), jnp.float32),
        pltpu.VMEM((CH, D), jnp.float32),
        pltpu.VMEM((16,), jnp.float32),
        pltpu.SemaphoreType.DMA,
        pltpu.SemaphoreType.DMA,
        pltpu.SemaphoreType.DMA,
        pltpu.SemaphoreType.DMA,
    ],
)
def _sc_logits(q_hbm, k_hbm, src_hbm, dst_hbm, logits_hbm, maxes_hbm,
               srcall, dstall, lall, qrows0, krows0, qrows1, krows1, mxbuf,
               qsem0, ksem0, qsem1, ksem1):
    wid = _worker_id()
    base = wid * EWP
    lanes = lax.iota(jnp.int32, 16)
    inv = jnp.float32(INV_SQRT_H)

    pltpu.sync_copy(src_hbm.at[pl.ds(base, EWP)], srcall)
    pltpu.sync_copy(dst_hbm.at[pl.ds(base, EWP)], dstall)

    qbufs = (qrows0, qrows1)
    kbufs = (krows0, krows1)
    qsems = (qsem0, qsem1)
    ksems = (ksem0, ksem1)

    def issue(t, b):
        pltpu.async_copy(q_hbm.at[dstall.at[pl.ds(t * CH, CH)]],
                         qbufs[b], qsems[b])
        pltpu.async_copy(k_hbm.at[srcall.at[pl.ds(t * CH, CH)]],
                         kbufs[b], ksems[b])

    def wait(t, b):
        pltpu.make_async_copy(q_hbm.at[dstall.at[pl.ds(t * CH, CH)]],
                              qbufs[b], qsems[b]).wait()
        pltpu.make_async_copy(k_hbm.at[srcall.at[pl.ds(t * CH, CH)]],
                              kbufs[b], ksems[b]).wait()

    issue(0, 0)

    def pair_body(tt, rmax):
        for b in range(2):
            t = tt * 2 + b
            nt = jnp.minimum(t + 1, NCH - 1)
            issue(nt, 1 - b)
            wait(t, b)
            qr = qbufs[b]
            kr = kbufs[b]

            def group_body(g, rmax):
                lg = jnp.zeros((16,), jnp.float32)
                for j in range(16):
                    r = g * 16 + j
                    acc = qr[r, pl.ds(0, 16)] * kr[r, pl.ds(0, 16)]
                    for blk in range(1, D // 16):
                        sl = pl.ds(blk * 16, 16)
                        acc = acc + qr[r, sl] * kr[r, sl]
                    dv = jnp.sum(acc)
                    lg = jnp.where(lanes == j, dv, lg)
                lg = lg * inv
                lall[pl.ds(t * CH + g * 16, 16)] = lg
                return jnp.maximum(rmax, lg)

            rmax = lax.fori_loop(0, CH // 16, group_body, rmax)
        return rmax

    rmax = lax.fori_loop(0, NCH // 2, pair_body,
                         jnp.full((16,), -1e30, jnp.float32))
    wait(NCH - 1, 0)
    pltpu.sync_copy(lall, logits_hbm.at[pl.ds(base, EWP)])
    mxbuf[...] = rmax
    pltpu.sync_copy(mxbuf, maxes_hbm.at[wid])


@functools.partial(
    pl.kernel,
    out_type=jax.ShapeDtypeStruct((NC, NP), jnp.float32),
    mesh=_mesh,
    compiler_params=_params,
    scratch_types=[
        pltpu.VMEM_SHARED((NP,), jnp.float32),
        pltpu.VMEM((NP,), jnp.float32),
        pltpu.VMEM((EWP,), jnp.int32),
        pltpu.VMEM((EWP,), jnp.float32),
        pltpu.VMEM((NW * 16,), jnp.float32),
        pltpu.VMEM((CH,), jnp.int32),
        pltpu.VMEM((CH,), jnp.float32),
    ],
)
def _sc_sums(logits_hbm, dst_hbm, maxes_hbm, sums_hbm,
             s_acc, zvm, dstall, lall, maxes_vm, dsti, echunk):
    cid = lax.axis_index("c")
    sid = lax.axis_index("s")
    wid = cid * NS + sid
    base = wid * EWP

    pltpu.sync_copy(maxes_hbm, maxes_vm)
    gmax = _global_max(maxes_vm)
    pltpu.sync_copy(dst_hbm.at[pl.ds(base, EWP)], dstall)
    pltpu.sync_copy(logits_hbm.at[pl.ds(base, EWP)], lall)

    @pl.when(sid == 0)
    def _zero():
        z16 = jnp.zeros((16,), jnp.float32)

        def zb(i, carry):
            zvm[pl.ds(i * 16, 16)] = z16
            return carry

        lax.fori_loop(0, NP // 16, zb, 0)
        pltpu.sync_copy(zvm, s_acc)

    plsc.subcore_barrier()

    def chunk_body(t, carry):
        for g in range(CH // 16):
            dsti[pl.ds(g * 16, 16)] = dstall[pl.ds(t * CH + g * 16, 16)]
            lv = lall[pl.ds(t * CH + g * 16, 16)]
            echunk[pl.ds(g * 16, 16)] = jnp.exp(lv - gmax)
        pltpu.sync_copy(echunk, s_acc.at[dsti], add=True)
        return carry

    lax.fori_loop(0, NCH, chunk_body, 0)
    plsc.subcore_barrier()

    @pl.when(sid == 0)
    def _dump():
        pltpu.sync_copy(s_acc, zvm)
        pltpu.sync_copy(zvm, sums_hbm.at[cid])


@functools.partial(
    pl.kernel,
    out_type=jax.ShapeDtypeStruct((EP,), jnp.float32),
    mesh=_mesh,
    compiler_params=_params,
    scratch_types=[
        pltpu.VMEM((EWP,), jnp.int32),
        pltpu.VMEM((EWP,), jnp.float32),
        pltpu.VMEM((NP,), jnp.float32),
        pltpu.VMEM((NW * 16,), jnp.float32),
        pltpu.VMEM((CH,), jnp.float32),
    ],
)
def _sc_alpha(logits_hbm, dst_hbm, maxes_hbm, sums_hbm, alpha_hbm,
              dstall, lall, s_full, maxes_vm, sbuf):
    wid = _worker_id()
    base = wid * EWP

    pltpu.sync_copy(maxes_hbm, maxes_vm)
    gmax = _global_max(maxes_vm)
    pltpu.sync_copy(dst_hbm.at[pl.ds(base, EWP)], dstall)
    pltpu.sync_copy(logits_hbm.at[pl.ds(base, EWP)], lall)

    pltpu.sync_copy(sums_hbm.at[0], s_full)
    eps = jnp.float32(1e-16)

    def comb(i, carry):
        pltpu.sync_copy(sums_hbm.at[1, pl.ds(i * CH, CH)], sbuf)
        for g in range(CH // 16):
            sl = pl.ds(i * CH + g * 16, 16)
            s_full[sl] = s_full[sl] + sbuf[pl.ds(g * 16, 16)] + eps
        return carry

    lax.fori_loop(0, NP // CH, comb, 0)

    def seg_body(i, carry):
        sl = pl.ds(i * 16, 16)
        dst16 = dstall[sl]
        sv = plsc.load_gather(s_full, [dst16])
        lall[sl] = jnp.exp(lall[sl] - gmax) / sv
        return carry

    lax.fori_loop(0, EWP // 16, seg_body, 0)
    pltpu.sync_copy(lall, alpha_hbm.at[pl.ds(base, EWP)])


@functools.partial(
    pl.kernel,
    out_type=jax.ShapeDtypeStruct((NC, NP, D), jnp.float32),
    mesh=_mesh,
    compiler_params=_params,
    scratch_types=[
        pltpu.VMEM_SHARED((NP, D), jnp.float32),
        pltpu.VMEM((CH,), jnp.int32),
        pltpu.VMEM((CH,), jnp.int32),
        pltpu.VMEM((CH,), jnp.float32),
        pltpu.VMEM((CH,), jnp.int32),
        pltpu.VMEM((CH,), jnp.int32),
        pltpu.VMEM((CH,), jnp.float32),
        pltpu.VMEM((CH, D), jnp.float32),
        pltpu.VMEM((CH, D), jnp.float32),
        pltpu.SemaphoreType.DMA,
        pltpu.SemaphoreType.DMA,
        pltpu.SemaphoreType.DMA,
        pltpu.SemaphoreType.DMA,
    ],
)
def _sc_aggregate(v_hbm, src_hbm, dst_hbm, alpha_hbm, parts_hbm,
                  out_acc, srci0, dsti0, alf0, srci1, dsti1, alf1,
                  vrows0, vrows1, msem0, msem1, gsem0, gsem1):
    cid = lax.axis_index("c")
    sid = lax.axis_index("s")
    wid = cid * NS + sid
    base = wid * EWP

    srcis = (srci0, srci1)
    dstis = (dsti0, dsti1)
    alfs = (alf0, alf1)
    vbufs = (vrows0, vrows1)
    msems = (msem0, msem1)
    gsems = (gsem0, gsem1)

    def meta_issue(t, b):
        off = base + t * CH
        pltpu.async_copy(src_hbm.at[pl.ds(off, CH)], srcis[b], msems[b])
        pltpu.async_copy(dst_hbm.at[pl.ds(off, CH)], dstis[b], msems[b])
        pltpu.async_copy(alpha_hbm.at[pl.ds(off, CH)], alfs[b], msems[b])

    def meta_wait(t, b):
        off = base + t * CH
        pltpu.make_async_copy(src_hbm.at[pl.ds(off, CH)],
                              srcis[b], msems[b]).wait()
        pltpu.make_async_copy(dst_hbm.at[pl.ds(off, CH)],
                              dstis[b], msems[b]).wait()
        pltpu.make_async_copy(alpha_hbm.at[pl.ds(off, CH)],
                              alfs[b], msems[b]).wait()

    def gather_issue(b):
        pltpu.async_copy(v_hbm.at[srcis[b]], vbufs[b], gsems[b])

    def gather_wait(b):
        pltpu.make_async_copy(v_hbm.at[srcis[b]], vbufs[b], gsems[b]).wait()

    @pl.when(sid == 0)
    def _zero():
        z16 = jnp.zeros((16,), jnp.float32)

        def zb(i, carry):
            for blk in range(D // 16):
                vrows0[i, pl.ds(blk * 16, 16)] = z16
            return carry

        lax.fori_loop(0, CH, zb, 0)

        def zcopy(i, carry):
            pltpu.sync_copy(vrows0, out_acc.at[pl.ds(i * CH, CH)])
            return carry

        lax.fori_loop(0, NP // CH, zcopy, 0)

    plsc.subcore_barrier()

    meta_issue(0, 0)
    meta_wait(0, 0)
    gather_issue(0)
    meta_issue(1, 1)

    def pair_body(tt, carry):
        for b in range(2):
            t = tt * 2 + b
            ob = 1 - b
            gather_wait(b)
            meta_wait(jnp.minimum(t + 1, NCH - 1), ob)
            gather_issue(ob)

            def group_scale(g, carry):
                a16 = alfs[b][pl.ds(g * 16, 16)]
                for j in range(16):
                    a = a16[j]
                    for blk in range(D // 16):
                        sl = pl.ds(blk * 16, 16)
                        vbufs[b][g * 16 + j, sl] = vbufs[b][g * 16 + j, sl] * a
                return carry

            lax.fori_loop(0, CH // 16, group_scale, 0)
            pltpu.sync_copy(vbufs[b], out_acc.at[dstis[b]], add=True)
            meta_issue(jnp.minimum(t + 2, NCH - 1), b)
        return carry

    lax.fori_loop(0, NCH // 2, pair_body, 0)
    gather_wait(0)
    meta_wait(NCH - 1, 1)
    plsc.subcore_barrier()

    pltpu.sync_copy(out_acc.at[pl.ds(sid * ROWS_W, ROWS_W)],
                    parts_hbm.at[cid, pl.ds(sid * ROWS_W, ROWS_W)])


_BLK = 2528


def _proj1_body(x_ref, w_ref, b_ref, q_ref, k_ref, v_ref, s_ref):
    r = jnp.dot(x_ref[...], w_ref[...],
                preferred_element_type=jnp.float32) + b_ref[...]
    q_ref[...] = r[:, 0 * D:1 * D]
    k_ref[...] = r[:, 1 * D:2 * D]
    v_ref[...] = r[:, 2 * D:3 * D]
    s_ref[...] = r[:, 3 * D:4 * D]


def _proj2_body(p0_ref, p1_ref, sk_ref, w_ref, b_ref,
                q_ref, k_ref, v_ref, s_ref):
    h = jnp.maximum(p0_ref[...] + p1_ref[...] + sk_ref[...], 0.0)
    r = jnp.dot(h, w_ref[...], preferred_element_type=jnp.float32) + b_ref[...]
    q_ref[...] = r[:, 0 * D:1 * D]
    k_ref[...] = r[:, 1 * D:2 * D]
    v_ref[...] = r[:, 2 * D:3 * D]
    s_ref[...] = r[:, 3 * D:4 * D]


def _combine_body(p0_ref, p1_ref, sk_ref, o_ref):
    o_ref[...] = p0_ref[...] + p1_ref[...] + sk_ref[...]


def _proj1(x, wcat, bcat):
    return pl.pallas_call(
        _proj1_body,
        grid=(NP // _BLK,),
        in_specs=[
            pl.BlockSpec((_BLK, D), lambda i: (i, 0)),
            pl.BlockSpec((D, 4 * D), lambda i: (0, 0)),
            pl.BlockSpec((1, 4 * D), lambda i: (0, 0)),
        ],
        out_specs=[pl.BlockSpec((_BLK, D), lambda i: (i, 0))] * 4,
        out_shape=[jax.ShapeDtypeStruct((NP, D), jnp.float32)] * 4,
    )(x, wcat, bcat)


def _proj2(p0, p1, sk, wcat, bcat):
    return pl.pallas_call(
        _proj2_body,
        grid=(NP // _BLK,),
        in_specs=[
            pl.BlockSpec((_BLK, D), lambda i: (i, 0)),
            pl.BlockSpec((_BLK, D), lambda i: (i, 0)),
            pl.BlockSpec((_BLK, D), lambda i: (i, 0)),
            pl.BlockSpec((D, 4 * D), lambda i: (0, 0)),
            pl.BlockSpec((1, 4 * D), lambda i: (0, 0)),
        ],
        out_specs=[pl.BlockSpec((_BLK, D), lambda i: (i, 0))] * 4,
        out_shape=[jax.ShapeDtypeStruct((NP, D), jnp.float32)] * 4,
    )(p0, p1, sk, wcat, bcat)


def _combine(p0, p1, sk):
    return pl.pallas_call(
        _combine_body,
        grid=(5,),
        in_specs=[pl.BlockSpec((N // 5, D), lambda i: (i, 0))] * 3,
        out_specs=pl.BlockSpec((N // 5, D), lambda i: (i, 0)),
        out_shape=jax.ShapeDtypeStruct((N, D), jnp.float32),
    )(p0, p1, sk)


def _edge_softmax_aggregate(q, k, v, srcp, dstp):
    logits, maxes = _sc_logits(q, k, srcp, dstp)
    maxes = maxes.reshape(-1)
    sums = _sc_sums(logits, dstp, maxes)
    alpha = _sc_alpha(logits, dstp, maxes, sums)
    parts = _sc_aggregate(v, srcp, dstp, alpha)
    return parts[0], parts[1]


def kernel(x, edge_index, Wq1, bq1, Wk1, bk1, Wv1, bv1, Ws1, bs1,
           Wq2, bq2, Wk2, bk2, Wv2, bv2, Ws2, bs2):
    src = edge_index[0]
    dst = edge_index[1]
    srcp = jnp.concatenate([src, jnp.zeros((EP - E,), jnp.int32)])
    dstp = jnp.concatenate([dst, jnp.full((EP - E,), N, jnp.int32)])
    xp = jnp.pad(x, ((0, NP - N), (0, 0)))

    w1 = jnp.concatenate([Wq1, Wk1, Wv1, Ws1], axis=1)
    b1 = jnp.concatenate([bq1, bk1, bv1, bs1]).reshape(1, -1)
    w2 = jnp.concatenate([Wq2, Wk2, Wv2, Ws2], axis=1)
    b2 = jnp.concatenate([bq2, bk2, bv2, bs2]).reshape(1, -1)

    q1, k1, v1, sk1 = _proj1(xp, w1, b1)
    p0, p1 = _edge_softmax_aggregate(q1, k1, v1, srcp, dstp)
    q2, k2, v2, sk2 = _proj2(p0, p1, sk1, w2, b2)
    p0, p1 = _edge_softmax_aggregate(q2, k2, v2, srcp, dstp)
    return _combine(p0[:N], p1[:N], sk2[:N])
